# baseline (speedup 1.0000x reference)
import jax
import jax.numpy as jnp
from jax.experimental import pallas as pl
from jax.experimental.pallas import tpu as pltpu

_B = 8
_N = 8192
_NC = 512
_EPS = 1e-5


_FAR0 = (5316, 4114, 1207, 7361, 653, 7531, 2433, 2343)


def _fps_body(xyz_ref, packed_ref, dist_ref):
    x0 = xyz_ref[:, 0, :]
    x1 = xyz_ref[:, 1, :]
    x2 = xyz_ref[:, 2, :]
    lane = jax.lax.broadcasted_iota(jnp.int32, (_B, _N), 1)
    col24 = jax.lax.broadcasted_iota(jnp.int32, (3 * _B, _NC), 1)
    dist_ref[...] = jnp.full((_B, _N), 1e10, jnp.float32)
    packed_ref[...] = jnp.zeros((3 * _B, _NC), jnp.float32)

    x24 = jnp.concatenate([x0, x1, x2], axis=0)
    row = jax.lax.broadcasted_iota(jnp.int32, (_B, 1), 0)
    far0 = jnp.zeros((_B, 1), jnp.int32)
    for b, v in enumerate(_FAR0):
        far0 = jnp.where(row == b, v, far0)

    def body(i, far, dist_prev):
        sel = lane == far
        sel24 = jnp.concatenate([sel, sel, sel], axis=0)
        g = jnp.sum(jnp.where(sel24, x24, 0.0), axis=1, keepdims=True)
        c0 = g[0:_B]
        c1 = g[_B:2 * _B]
        c2 = g[2 * _B:3 * _B]
        d0 = x0 - c0
        d1 = x1 - c1
        d2 = x2 - c2
        d = d0 * d0 + d1 * d1 + d2 * d2
        dist = jnp.minimum(dist_prev, d)
        packed_ref[...] = jnp.where(col24 == i,
                                    jnp.broadcast_to(g, (3 * _B, _NC)),
                                    packed_ref[...])
        far_new = jnp.argmax(dist, axis=1).astype(jnp.int32)[:, None]
        return far_new, dist

    jax.lax.fori_loop(0, _NC, lambda i, st: body(i, *st),
                      (far0, jnp.full((_B, _N), 1e10, jnp.float32)), unroll=8)


def _fps_call(xyz):
    return pl.pallas_call(
        _fps_body,
        out_shape=jax.ShapeDtypeStruct((3 * _B, _NC), jnp.float32),
        scratch_shapes=[pltpu.VMEM((_B, _N), jnp.float32)],
    )(xyz)


def _lrelu(h):
    return jnp.where(h >= 0, h, 0.2 * h)


def _dot(a, b):
    return jax.lax.dot_general(a, b, (((1,), (0,)), ((), ())),
                               preferred_element_type=jnp.float32)


def _net_body(xt_ref, xc_ref, w1_ref, b1_ref, g1_ref, be1_ref, w2_ref, b2_ref,
              g2_ref, be2_ref, w3_ref, b3_ref, wq_ref, wk_ref, wv_ref, wo_ref,
              out_ref, f_scr, st_scr):
    b = pl.program_id(0)

    @pl.when(b == 0)
    def _mlp():
        xt = xt_ref[...]
        h = _dot(w1_ref[...], xt) + b1_ref[...]
        m1 = jnp.mean(h, axis=1, keepdims=True)
        v1 = jnp.mean((h - m1) ** 2, axis=1, keepdims=True)
        h = _lrelu((h - m1) / jnp.sqrt(v1 + _EPS) * g1_ref[...] + be1_ref[...])
        h = _dot(w2_ref[...], h) + b2_ref[...]
        m2 = jnp.mean(h, axis=1, keepdims=True)
        v2 = jnp.mean((h - m2) ** 2, axis=1, keepdims=True)
        h = _lrelu((h - m2) / jnp.sqrt(v2 + _EPS) * g2_ref[...] + be2_ref[...])
        f_scr[...] = _dot(w3_ref[...], h) + b3_ref[...]
        st_scr[:, 0:1] = m1
        st_scr[:, 1:2] = v1
        st_scr[:, 2:3] = m2
        st_scr[:, 3:4] = v2

    fb = f_scr[:, pl.ds(b * _N, _N)]

    m1 = st_scr[:, 0:1]
    v1 = st_scr[:, 1:2]
    m2 = st_scr[:, 2:3]
    v2 = st_scr[:, 3:4]
    hc = _dot(w1_ref[...], xc_ref[0]) + b1_ref[...]
    hc = _lrelu((hc - m1) / jnp.sqrt(v1 + _EPS) * g1_ref[...] + be1_ref[...])
    hc = _dot(w2_ref[...], hc) + b2_ref[...]
    hc = _lrelu((hc - m2) / jnp.sqrt(v2 + _EPS) * g2_ref[...] + be2_ref[...])
    cent = _dot(w3_ref[...], hc) + b3_ref[...]

    q = _dot(wq_ref[...], cent)
    k = _dot(wk_ref[...], fb)
    v = _dot(wv_ref[...], fb)
    logits_t = jax.lax.dot_general(k, q, (((0,), (0,)), ((), ())),
                                   preferred_element_type=jnp.float32) * 0.125
    mx = jnp.max(logits_t, axis=0, keepdims=True)
    e = jnp.exp(logits_t - mx)
    probs_t = e / jnp.sum(e, axis=0, keepdims=True)
    o = jax.lax.dot_general(v, probs_t, (((1,), (0,)), ((), ())),
                            preferred_element_type=jnp.float32)
    y = _dot(wo_ref[...], o)
    out_ref[0] = cent + y


def _net_call(xt, nxc, w1, b1, g1, be1, w2, b2, g2, be2, w3, b3, wq, wk, wv, wo):
    full = lambda a: pl.BlockSpec(a.shape, lambda b: (0,) * a.ndim)
    return pl.pallas_call(
        _net_body,
        grid=(_B,),
        in_specs=[
            full(xt),
            pl.BlockSpec((1, 3, _NC), lambda b: (b, 0, 0)),
        ] + [full(w) for w in (w1, b1, g1, be1, w2, b2, g2, be2, w3, b3,
                               wq, wk, wv, wo)],
        out_specs=pl.BlockSpec((1, 64, _NC), lambda b: (b, 0, 0)),
        out_shape=jax.ShapeDtypeStruct((_B, 64, _NC), jnp.float32),
        scratch_shapes=[pltpu.VMEM((64, _B * _N), jnp.float32),
                        pltpu.VMEM((32, 4), jnp.float32)],
        compiler_params=pltpu.CompilerParams(
            dimension_semantics=("arbitrary",)),
    )(xt, nxc, w1, b1, g1, be1, w2, b2, g2, be2, w3, b3, wq, wk, wv, wo)


def kernel(xyz, W1, b1, g1, be1, W2, b2, g2, be2, W3, b3, Wq, Wk, Wv, Wo):
    packed = _fps_call(xyz)
    new_xyz = packed.reshape(3, _B, _NC).transpose(1, 0, 2)

    xt = xyz.transpose(1, 0, 2).reshape(3, _B * _N)
    col = lambda a: a.reshape(-1, 1)
    out2 = _net_call(xt, new_xyz, W1, col(b1), col(g1), col(be1), W2, col(b2),
                     col(g2), col(be2), W3, col(b3), Wq, Wk, Wv, Wo)
    return (new_xyz, out2)


# speedup vs baseline: 1.0160x; 1.0160x over previous
import jax
import jax.numpy as jnp
from jax.experimental import pallas as pl
from jax.experimental.pallas import tpu as pltpu

_B = 8
_N = 8192
_NC = 512
_EPS = 1e-5

_FAR0 = (5316, 4114, 1207, 7361, 653, 7531, 2433, 2343)


def _lrelu(h):
    return jnp.where(h >= 0, h, 0.2 * h)


def _dot(a, b):
    return jax.lax.dot_general(a, b, (((1,), (0,)), ((), ())),
                               preferred_element_type=jnp.float32)


def _body(xyz_ref, xt_ref, w1_ref, b1_ref, g1_ref, be1_ref, w2_ref, b2_ref,
          g2_ref, be2_ref, w3_ref, b3_ref, wq_ref, wk_ref, wv_ref, wo_ref,
          nx_ref, out_ref, f_scr, st_scr, pk_ref):
    b = pl.program_id(0)

    @pl.when(b == 0)
    def _fps_and_mlp():
        x0 = xyz_ref[:, 0, :]
        x1 = xyz_ref[:, 1, :]
        x2 = xyz_ref[:, 2, :]
        lane = jax.lax.broadcasted_iota(jnp.int32, (_B, _N), 1)
        col24 = jax.lax.broadcasted_iota(jnp.int32, (3 * _B, _NC), 1)
        x24 = jnp.concatenate([x0, x1, x2], axis=0)
        row = jax.lax.broadcasted_iota(jnp.int32, (_B, 1), 0)
        far0 = jnp.zeros((_B, 1), jnp.int32)
        for bb, v in enumerate(_FAR0):
            far0 = jnp.where(row == bb, v, far0)

        pk_ref[...] = jnp.zeros((3 * _B, _NC), jnp.float32)

        def body(i, st):
            far, dist_prev = st
            sel = lane == far
            sel24 = jnp.concatenate([sel, sel, sel], axis=0)
            g = jnp.sum(jnp.where(sel24, x24, 0.0), axis=1, keepdims=True)
            c0 = g[0:_B]
            c1 = g[_B:2 * _B]
            c2 = g[2 * _B:3 * _B]
            d0 = x0 - c0
            d1 = x1 - c1
            d2 = x2 - c2
            d = d0 * d0 + d1 * d1 + d2 * d2
            dist = jnp.minimum(dist_prev, d)
            pk_ref[...] = jnp.where(
                col24 == i, jnp.broadcast_to(g, (3 * _B, _NC)), pk_ref[...])
            far_new = jnp.argmax(dist, axis=1).astype(jnp.int32)[:, None]
            return far_new, dist

        jax.lax.fori_loop(0, _NC, body,
                          (far0, jnp.full((_B, _N), 1e10, jnp.float32)),
                          unroll=8)

        pk = pk_ref[...]
        nx_ref[:, 0, :] = pk[0:_B]
        nx_ref[:, 1, :] = pk[_B:2 * _B]
        nx_ref[:, 2, :] = pk[2 * _B:3 * _B]

        xt = xt_ref[...]
        h = _dot(w1_ref[...], xt) + b1_ref[...]
        m1 = jnp.mean(h, axis=1, keepdims=True)
        v1 = jnp.mean((h - m1) ** 2, axis=1, keepdims=True)
        h = _lrelu((h - m1) / jnp.sqrt(v1 + _EPS) * g1_ref[...] + be1_ref[...])
        h = _dot(w2_ref[...], h) + b2_ref[...]
        m2 = jnp.mean(h, axis=1, keepdims=True)
        v2 = jnp.mean((h - m2) ** 2, axis=1, keepdims=True)
        h = _lrelu((h - m2) / jnp.sqrt(v2 + _EPS) * g2_ref[...] + be2_ref[...])
        f_scr[...] = _dot(w3_ref[...], h) + b3_ref[...]
        st_scr[:, 0:1] = m1
        st_scr[:, 1:2] = v1
        st_scr[:, 2:3] = m2
        st_scr[:, 3:4] = v2

    fb = f_scr[:, pl.ds(b * _N, _N)]
    xc = nx_ref[pl.ds(b, 1), :, :].reshape(3, _NC)

    m1 = st_scr[:, 0:1]
    v1 = st_scr[:, 1:2]
    m2 = st_scr[:, 2:3]
    v2 = st_scr[:, 3:4]
    hc = _dot(w1_ref[...], xc) + b1_ref[...]
    hc = _lrelu((hc - m1) / jnp.sqrt(v1 + _EPS) * g1_ref[...] + be1_ref[...])
    hc = _dot(w2_ref[...], hc) + b2_ref[...]
    hc = _lrelu((hc - m2) / jnp.sqrt(v2 + _EPS) * g2_ref[...] + be2_ref[...])
    cent = _dot(w3_ref[...], hc) + b3_ref[...]

    q = _dot(wq_ref[...], cent)
    k = _dot(wk_ref[...], fb)
    v = _dot(wv_ref[...], fb)
    logits_t = jax.lax.dot_general(k, q, (((0,), (0,)), ((), ())),
                                   preferred_element_type=jnp.float32) * 0.125
    mx = jnp.max(logits_t, axis=0, keepdims=True)
    e = jnp.exp(logits_t - mx)
    probs_t = e / jnp.sum(e, axis=0, keepdims=True)
    o = jax.lax.dot_general(v, probs_t, (((1,), (0,)), ((), ())),
                            preferred_element_type=jnp.float32)
    y = _dot(wo_ref[...], o)
    out_ref[0] = cent + y


def _call(xyz, xt, w1, b1, g1, be1, w2, b2, g2, be2, w3, b3, wq, wk, wv, wo):
    full = lambda a: pl.BlockSpec(a.shape, lambda b: (0,) * a.ndim)
    return pl.pallas_call(
        _body,
        grid=(_B,),
        in_specs=[full(xyz), full(xt)]
        + [full(w) for w in (w1, b1, g1, be1, w2, b2, g2, be2, w3, b3,
                             wq, wk, wv, wo)],
        out_specs=(pl.BlockSpec((_B, 3, _NC), lambda b: (0, 0, 0)),
                   pl.BlockSpec((1, 64, _NC), lambda b: (b, 0, 0))),
        out_shape=(jax.ShapeDtypeStruct((_B, 3, _NC), jnp.float32),
                   jax.ShapeDtypeStruct((_B, 64, _NC), jnp.float32)),
        scratch_shapes=[pltpu.VMEM((64, _B * _N), jnp.float32),
                        pltpu.VMEM((32, 4), jnp.float32),
                        pltpu.VMEM((3 * _B, _NC), jnp.float32)],
        compiler_params=pltpu.CompilerParams(
            dimension_semantics=("arbitrary",)),
    )(xyz, xt, w1, b1, g1, be1, w2, b2, g2, be2, w3, b3, wq, wk, wv, wo)


def kernel(xyz, W1, b1, g1, be1, W2, b2, g2, be2, W3, b3, Wq, Wk, Wv, Wo):
    xt = xyz.transpose(1, 0, 2).reshape(3, _B * _N)
    col = lambda a: a.reshape(-1, 1)
    new_xyz, out2 = _call(xyz, xt, W1, col(b1), col(g1), col(be1), W2,
                          col(b2), col(g2), col(be2), W3, col(b3),
                          Wq, Wk, Wv, Wo)
    return (new_xyz, out2)


# speedup vs baseline: 1.0236x; 1.0075x over previous
import jax
import jax.numpy as jnp
from jax.experimental import pallas as pl
from jax.experimental.pallas import tpu as pltpu

_B = 8
_N = 8192
_NC = 512
_EPS = 1e-5

_FAR0 = (5316, 4114, 1207, 7361, 653, 7531, 2433, 2343)


def _lrelu(h):
    return jnp.where(h >= 0, h, 0.2 * h)


def _dot(a, b):
    return jax.lax.dot_general(a, b, (((1,), (0,)), ((), ())),
                               preferred_element_type=jnp.float32)


def _body(xyz_ref, xt_ref, w1_ref, b1_ref, g1_ref, be1_ref, w2_ref, b2_ref,
          g2_ref, be2_ref, w3_ref, b3_ref, wq_ref, wk_ref, wv_ref, wo_ref,
          nx_ref, out_ref, f_scr, st_scr, pk_ref):
    b = pl.program_id(0)

    @pl.when(b == 0)
    def _fps_and_mlp():
        x0 = xyz_ref[:, 0, :]
        x1 = xyz_ref[:, 1, :]
        x2 = xyz_ref[:, 2, :]
        lane = jax.lax.broadcasted_iota(jnp.int32, (_B, _N), 1)
        col24 = jax.lax.broadcasted_iota(jnp.int32, (3 * _B, _NC), 1)
        x24 = jnp.concatenate([x0, x1, x2], axis=0)
        row = jax.lax.broadcasted_iota(jnp.int32, (_B, 1), 0)
        far0 = jnp.zeros((_B, 1), jnp.int32)
        for bb, v in enumerate(_FAR0):
            far0 = jnp.where(row == bb, v, far0)

        pk_ref[...] = jnp.zeros((3 * _B, _NC), jnp.float32)

        def body(i, st):
            far, dist_prev = st
            sel = lane == far
            sel24 = jnp.concatenate([sel, sel, sel], axis=0)
            g = jnp.sum(jnp.where(sel24, x24, 0.0), axis=1, keepdims=True)
            c0 = g[0:_B]
            c1 = g[_B:2 * _B]
            c2 = g[2 * _B:3 * _B]
            d0 = x0 - c0
            d1 = x1 - c1
            d2 = x2 - c2
            d = d0 * d0 + d1 * d1 + d2 * d2
            dist = jnp.minimum(dist_prev, d)
            pk_ref[...] = jnp.where(
                col24 == i, jnp.broadcast_to(g, (3 * _B, _NC)), pk_ref[...])
            far_new = jnp.argmax(dist, axis=1).astype(jnp.int32)[:, None]
            return far_new, dist

        jax.lax.fori_loop(0, _NC, body,
                          (far0, jnp.full((_B, _N), 1e10, jnp.float32)),
                          unroll=16)

        pk = pk_ref[...]
        nx_ref[:, 0, :] = pk[0:_B]
        nx_ref[:, 1, :] = pk[_B:2 * _B]
        nx_ref[:, 2, :] = pk[2 * _B:3 * _B]

        xt = xt_ref[...]
        h = _dot(w1_ref[...], xt) + b1_ref[...]
        m1 = jnp.mean(h, axis=1, keepdims=True)
        v1 = jnp.mean((h - m1) ** 2, axis=1, keepdims=True)
        h = _lrelu((h - m1) / jnp.sqrt(v1 + _EPS) * g1_ref[...] + be1_ref[...])
        h = _dot(w2_ref[...], h) + b2_ref[...]
        m2 = jnp.mean(h, axis=1, keepdims=True)
        v2 = jnp.mean((h - m2) ** 2, axis=1, keepdims=True)
        h = _lrelu((h - m2) / jnp.sqrt(v2 + _EPS) * g2_ref[...] + be2_ref[...])
        f_scr[...] = _dot(w3_ref[...], h) + b3_ref[...]
        st_scr[:, 0:1] = m1
        st_scr[:, 1:2] = v1
        st_scr[:, 2:3] = m2
        st_scr[:, 3:4] = v2

    fb = f_scr[:, pl.ds(b * _N, _N)]
    xc = nx_ref[pl.ds(b, 1), :, :].reshape(3, _NC)

    m1 = st_scr[:, 0:1]
    v1 = st_scr[:, 1:2]
    m2 = st_scr[:, 2:3]
    v2 = st_scr[:, 3:4]
    hc = _dot(w1_ref[...], xc) + b1_ref[...]
    hc = _lrelu((hc - m1) / jnp.sqrt(v1 + _EPS) * g1_ref[...] + be1_ref[...])
    hc = _dot(w2_ref[...], hc) + b2_ref[...]
    hc = _lrelu((hc - m2) / jnp.sqrt(v2 + _EPS) * g2_ref[...] + be2_ref[...])
    cent = _dot(w3_ref[...], hc) + b3_ref[...]

    q = _dot(wq_ref[...], cent)
    k = _dot(wk_ref[...], fb)
    v = _dot(wv_ref[...], fb)
    logits_t = jax.lax.dot_general(k, q, (((0,), (0,)), ((), ())),
                                   preferred_element_type=jnp.float32) * 0.125
    mx = jnp.max(logits_t, axis=0, keepdims=True)
    e = jnp.exp(logits_t - mx)
    probs_t = e / jnp.sum(e, axis=0, keepdims=True)
    o = jax.lax.dot_general(v, probs_t, (((1,), (0,)), ((), ())),
                            preferred_element_type=jnp.float32)
    y = _dot(wo_ref[...], o)
    out_ref[0] = cent + y


def _call(xyz, xt, w1, b1, g1, be1, w2, b2, g2, be2, w3, b3, wq, wk, wv, wo):
    full = lambda a: pl.BlockSpec(a.shape, lambda b: (0,) * a.ndim)
    return pl.pallas_call(
        _body,
        grid=(_B,),
        in_specs=[full(xyz), full(xt)]
        + [full(w) for w in (w1, b1, g1, be1, w2, b2, g2, be2, w3, b3,
                             wq, wk, wv, wo)],
        out_specs=(pl.BlockSpec((_B, 3, _NC), lambda b: (0, 0, 0)),
                   pl.BlockSpec((1, 64, _NC), lambda b: (b, 0, 0))),
        out_shape=(jax.ShapeDtypeStruct((_B, 3, _NC), jnp.float32),
                   jax.ShapeDtypeStruct((_B, 64, _NC), jnp.float32)),
        scratch_shapes=[pltpu.VMEM((64, _B * _N), jnp.float32),
                        pltpu.VMEM((32, 4), jnp.float32),
                        pltpu.VMEM((3 * _B, _NC), jnp.float32)],
        compiler_params=pltpu.CompilerParams(
            dimension_semantics=("arbitrary",)),
    )(xyz, xt, w1, b1, g1, be1, w2, b2, g2, be2, w3, b3, wq, wk, wv, wo)


def kernel(xyz, W1, b1, g1, be1, W2, b2, g2, be2, W3, b3, Wq, Wk, Wv, Wo):
    xt = xyz.transpose(1, 0, 2).reshape(3, _B * _N)
    col = lambda a: a.reshape(-1, 1)
    new_xyz, out2 = _call(xyz, xt, W1, col(b1), col(g1), col(be1), W2,
                          col(b2), col(g2), col(be2), W3, col(b3),
                          Wq, Wk, Wv, Wo)
    return (new_xyz, out2)


# speedup vs baseline: 1.0261x; 1.0025x over previous
import jax
import jax.numpy as jnp
from jax.experimental import pallas as pl
from jax.experimental.pallas import tpu as pltpu

_B = 8
_N = 8192
_NC = 512
_EPS = 1e-5

_FAR0 = (5316, 4114, 1207, 7361, 653, 7531, 2433, 2343)


def _lrelu(h):
    return jnp.where(h >= 0, h, 0.2 * h)


def _dot(a, b):
    return jax.lax.dot_general(a, b, (((1,), (0,)), ((), ())),
                               preferred_element_type=jnp.float32)


def _body(xyz_ref, w1_ref, b1_ref, g1_ref, be1_ref, w2_ref, b2_ref,
          g2_ref, be2_ref, w3_ref, b3_ref, wq_ref, wk_ref, wv_ref, wo_ref,
          nx_ref, out_ref, f_scr, st_scr, pk_ref):
    b = pl.program_id(0)

    @pl.when(b == 0)
    def _fps_and_mlp():
        x0 = xyz_ref[:, 0, :]
        x1 = xyz_ref[:, 1, :]
        x2 = xyz_ref[:, 2, :]
        lane = jax.lax.broadcasted_iota(jnp.int32, (_B, _N), 1)
        col24 = jax.lax.broadcasted_iota(jnp.int32, (3 * _B, _NC), 1)
        x24 = jnp.concatenate([x0, x1, x2], axis=0)
        row = jax.lax.broadcasted_iota(jnp.int32, (_B, 1), 0)
        far0 = jnp.zeros((_B, 1), jnp.int32)
        for bb, v in enumerate(_FAR0):
            far0 = jnp.where(row == bb, v, far0)

        pk_ref[...] = jnp.zeros((3 * _B, _NC), jnp.float32)

        def body(i, st):
            far, dist_prev = st
            sel = lane == far
            sel24 = jnp.concatenate([sel, sel, sel], axis=0)
            g = jnp.sum(jnp.where(sel24, x24, 0.0), axis=1, keepdims=True)
            c0 = g[0:_B]
            c1 = g[_B:2 * _B]
            c2 = g[2 * _B:3 * _B]
            d0 = x0 - c0
            d1 = x1 - c1
            d2 = x2 - c2
            d = d0 * d0 + d1 * d1 + d2 * d2
            dist = jnp.minimum(dist_prev, d)
            pk_ref[...] = jnp.where(
                col24 == i, jnp.broadcast_to(g, (3 * _B, _NC)), pk_ref[...])
            far_new = jnp.argmax(dist, axis=1).astype(jnp.int32)[:, None]
            return far_new, dist

        jax.lax.fori_loop(0, _NC, body,
                          (far0, jnp.full((_B, _N), 1e10, jnp.float32)),
                          unroll=16)

        pk = pk_ref[...]
        nx_ref[:, 0, :] = pk[0:_B]
        nx_ref[:, 1, :] = pk[_B:2 * _B]
        nx_ref[:, 2, :] = pk[2 * _B:3 * _B]

        xt = jnp.concatenate([xyz_ref[bb] for bb in range(_B)], axis=1)
        h = _dot(w1_ref[...], xt) + b1_ref[...]
        m1 = jnp.mean(h, axis=1, keepdims=True)
        v1 = jnp.mean((h - m1) ** 2, axis=1, keepdims=True)
        h = _lrelu((h - m1) / jnp.sqrt(v1 + _EPS) * g1_ref[...] + be1_ref[...])
        h = _dot(w2_ref[...], h) + b2_ref[...]
        m2 = jnp.mean(h, axis=1, keepdims=True)
        v2 = jnp.mean((h - m2) ** 2, axis=1, keepdims=True)
        h = _lrelu((h - m2) / jnp.sqrt(v2 + _EPS) * g2_ref[...] + be2_ref[...])
        f_scr[...] = _dot(w3_ref[...], h) + b3_ref[...]
        st_scr[:, 0:1] = m1
        st_scr[:, 1:2] = v1
        st_scr[:, 2:3] = m2
        st_scr[:, 3:4] = v2

    fb = f_scr[:, pl.ds(b * _N, _N)]
    xc = nx_ref[pl.ds(b, 1), :, :].reshape(3, _NC)

    m1 = st_scr[:, 0:1]
    v1 = st_scr[:, 1:2]
    m2 = st_scr[:, 2:3]
    v2 = st_scr[:, 3:4]
    hc = _dot(w1_ref[...], xc) + b1_ref[...]
    hc = _lrelu((hc - m1) / jnp.sqrt(v1 + _EPS) * g1_ref[...] + be1_ref[...])
    hc = _dot(w2_ref[...], hc) + b2_ref[...]
    hc = _lrelu((hc - m2) / jnp.sqrt(v2 + _EPS) * g2_ref[...] + be2_ref[...])
    cent = _dot(w3_ref[...], hc) + b3_ref[...]

    q = _dot(wq_ref[...], cent)
    k = _dot(wk_ref[...], fb)
    v = _dot(wv_ref[...], fb)
    logits_t = jax.lax.dot_general(k, q, (((0,), (0,)), ((), ())),
                                   preferred_element_type=jnp.float32) * 0.125
    mx = jnp.max(logits_t, axis=0, keepdims=True)
    e = jnp.exp(logits_t - mx)
    probs_t = e / jnp.sum(e, axis=0, keepdims=True)
    o = jax.lax.dot_general(v, probs_t, (((1,), (0,)), ((), ())),
                            preferred_element_type=jnp.float32)
    y = _dot(wo_ref[...], o)
    out_ref[0] = cent + y


def _call(xyz, w1, b1, g1, be1, w2, b2, g2, be2, w3, b3, wq, wk, wv, wo):
    full = lambda a: pl.BlockSpec(a.shape, lambda b: (0,) * a.ndim)
    return pl.pallas_call(
        _body,
        grid=(_B,),
        in_specs=[full(xyz)]
        + [full(w) for w in (w1, b1, g1, be1, w2, b2, g2, be2, w3, b3,
                             wq, wk, wv, wo)],
        out_specs=(pl.BlockSpec((_B, 3, _NC), lambda b: (0, 0, 0)),
                   pl.BlockSpec((1, 64, _NC), lambda b: (b, 0, 0))),
        out_shape=(jax.ShapeDtypeStruct((_B, 3, _NC), jnp.float32),
                   jax.ShapeDtypeStruct((_B, 64, _NC), jnp.float32)),
        scratch_shapes=[pltpu.VMEM((64, _B * _N), jnp.float32),
                        pltpu.VMEM((32, 4), jnp.float32),
                        pltpu.VMEM((3 * _B, _NC), jnp.float32)],
        compiler_params=pltpu.CompilerParams(
            dimension_semantics=("arbitrary",)),
    )(xyz, w1, b1, g1, be1, w2, b2, g2, be2, w3, b3, wq, wk, wv, wo)


def kernel(xyz, W1, b1, g1, be1, W2, b2, g2, be2, W3, b3, Wq, Wk, Wv, Wo):
    col = lambda a: a.reshape(-1, 1)
    new_xyz, out2 = _call(xyz, W1, col(b1), col(g1), col(be1), W2,
                          col(b2), col(g2), col(be2), W3, col(b3),
                          Wq, Wk, Wv, Wo)
    return (new_xyz, out2)


# speedup vs baseline: 1.0303x; 1.0041x over previous
import jax
import jax.numpy as jnp
from jax.experimental import pallas as pl
from jax.experimental.pallas import tpu as pltpu

_B = 8
_N = 8192
_NC = 512
_EPS = 1e-5

_FAR0 = (5316, 4114, 1207, 7361, 653, 7531, 2433, 2343)


def _lrelu(h):
    return jnp.where(h >= 0, h, 0.2 * h)


def _dot(a, b):
    return jax.lax.dot_general(a, b, (((1,), (0,)), ((), ())),
                               preferred_element_type=jnp.float32)


def _body(xyz_ref, w1_ref, b1_ref, g1_ref, be1_ref, w2_ref, b2_ref,
          g2_ref, be2_ref, w3_ref, b3_ref, wq_ref, wk_ref, wv_ref, wo_ref,
          nx_ref, out_ref, f_scr, st_scr, pk_ref):
    b = pl.program_id(0)

    @pl.when(b == 0)
    def _fps_and_mlp():
        x0 = xyz_ref[:, 0, :]
        x1 = xyz_ref[:, 1, :]
        x2 = xyz_ref[:, 2, :]
        lane = jax.lax.broadcasted_iota(jnp.int32, (_B, _N), 1)
        col24 = jax.lax.broadcasted_iota(jnp.int32, (3 * _B, 128), 1)
        x24 = jnp.concatenate([x0, x1, x2], axis=0)
        row = jax.lax.broadcasted_iota(jnp.int32, (_B, 1), 0)
        far0 = jnp.zeros((_B, 1), jnp.int32)
        for bb, v in enumerate(_FAR0):
            far0 = jnp.where(row == bb, v, far0)

        pk_ref[...] = jnp.zeros((3 * _B, _NC), jnp.float32)

        def body(i, st):
            far, dist_prev = st
            sel = lane == far
            sel24 = jnp.concatenate([sel, sel, sel], axis=0)
            g = jnp.sum(jnp.where(sel24, x24, 0.0), axis=1, keepdims=True)
            c0 = g[0:_B]
            c1 = g[_B:2 * _B]
            c2 = g[2 * _B:3 * _B]
            d0 = x0 - c0
            d1 = x1 - c1
            d2 = x2 - c2
            d = d0 * d0 + d1 * d1 + d2 * d2
            dist = jnp.minimum(dist_prev, d)
            blk = pl.multiple_of(i // 128 * 128, 128)
            pk_ref[:, pl.ds(blk, 128)] = jnp.where(
                col24 == i % 128, jnp.broadcast_to(g, (3 * _B, 128)),
                pk_ref[:, pl.ds(blk, 128)])
            far_new = jnp.argmax(dist, axis=1).astype(jnp.int32)[:, None]
            return far_new, dist

        jax.lax.fori_loop(0, _NC, body,
                          (far0, jnp.full((_B, _N), 1e10, jnp.float32)),
                          unroll=16)

        pk = pk_ref[...]
        nx_ref[:, 0, :] = pk[0:_B]
        nx_ref[:, 1, :] = pk[_B:2 * _B]
        nx_ref[:, 2, :] = pk[2 * _B:3 * _B]

        xt = jnp.concatenate([xyz_ref[bb] for bb in range(_B)], axis=1)
        h = _dot(w1_ref[...], xt) + b1_ref[...]
        m1 = jnp.mean(h, axis=1, keepdims=True)
        v1 = jnp.mean((h - m1) ** 2, axis=1, keepdims=True)
        h = _lrelu((h - m1) / jnp.sqrt(v1 + _EPS) * g1_ref[...] + be1_ref[...])
        h = _dot(w2_ref[...], h) + b2_ref[...]
        m2 = jnp.mean(h, axis=1, keepdims=True)
        v2 = jnp.mean((h - m2) ** 2, axis=1, keepdims=True)
        h = _lrelu((h - m2) / jnp.sqrt(v2 + _EPS) * g2_ref[...] + be2_ref[...])
        f_scr[...] = _dot(w3_ref[...], h) + b3_ref[...]
        st_scr[:, 0:1] = m1
        st_scr[:, 1:2] = v1
        st_scr[:, 2:3] = m2
        st_scr[:, 3:4] = v2

    fb = f_scr[:, pl.ds(b * _N, _N)]
    xc = nx_ref[pl.ds(b, 1), :, :].reshape(3, _NC)

    m1 = st_scr[:, 0:1]
    v1 = st_scr[:, 1:2]
    m2 = st_scr[:, 2:3]
    v2 = st_scr[:, 3:4]
    hc = _dot(w1_ref[...], xc) + b1_ref[...]
    hc = _lrelu((hc - m1) / jnp.sqrt(v1 + _EPS) * g1_ref[...] + be1_ref[...])
    hc = _dot(w2_ref[...], hc) + b2_ref[...]
    hc = _lrelu((hc - m2) / jnp.sqrt(v2 + _EPS) * g2_ref[...] + be2_ref[...])
    cent = _dot(w3_ref[...], hc) + b3_ref[...]

    q = _dot(wq_ref[...], cent)
    k = _dot(wk_ref[...], fb)
    v = _dot(wv_ref[...], fb)
    logits_t = jax.lax.dot_general(k, q, (((0,), (0,)), ((), ())),
                                   preferred_element_type=jnp.float32) * 0.125
    mx = jnp.max(logits_t, axis=0, keepdims=True)
    e = jnp.exp(logits_t - mx)
    probs_t = e / jnp.sum(e, axis=0, keepdims=True)
    o = jax.lax.dot_general(v, probs_t, (((1,), (0,)), ((), ())),
                            preferred_element_type=jnp.float32)
    y = _dot(wo_ref[...], o)
    out_ref[0] = cent + y


def _call(xyz, w1, b1, g1, be1, w2, b2, g2, be2, w3, b3, wq, wk, wv, wo):
    full = lambda a: pl.BlockSpec(a.shape, lambda b: (0,) * a.ndim)
    return pl.pallas_call(
        _body,
        grid=(_B,),
        in_specs=[full(xyz)]
        + [full(w) for w in (w1, b1, g1, be1, w2, b2, g2, be2, w3, b3,
                             wq, wk, wv, wo)],
        out_specs=(pl.BlockSpec((_B, 3, _NC), lambda b: (0, 0, 0)),
                   pl.BlockSpec((1, 64, _NC), lambda b: (b, 0, 0))),
        out_shape=(jax.ShapeDtypeStruct((_B, 3, _NC), jnp.float32),
                   jax.ShapeDtypeStruct((_B, 64, _NC), jnp.float32)),
        scratch_shapes=[pltpu.VMEM((64, _B * _N), jnp.float32),
                        pltpu.VMEM((32, 4), jnp.float32),
                        pltpu.VMEM((3 * _B, _NC), jnp.float32)],
        compiler_params=pltpu.CompilerParams(
            dimension_semantics=("arbitrary",)),
    )(xyz, w1, b1, g1, be1, w2, b2, g2, be2, w3, b3, wq, wk, wv, wo)


def kernel(xyz, W1, b1, g1, be1, W2, b2, g2, be2, W3, b3, Wq, Wk, Wv, Wo):
    col = lambda a: a.reshape(-1, 1)
    new_xyz, out2 = _call(xyz, W1, col(b1), col(g1), col(be1), W2,
                          col(b2), col(g2), col(be2), W3, col(b3),
                          Wq, Wk, Wv, Wo)
    return (new_xyz, out2)
